# baseline (speedup 1.0000x reference)
import functools

import jax
import jax.numpy as jnp
from jax import lax
from jax.experimental import pallas as pl
from jax.experimental.pallas import tpu as pltpu
from jax.experimental.pallas import tpu_sc as plsc

B = 16384
D = 64
NUM_TABLES = 4

_NC, _NS = 2, 16
_NW = _NC * _NS
_BPW = B // _NW
_CH = 128
_NCHUNK = _BPW // _CH


def _gather_body(type_hbm, loc_hbm, src_hbm, tgt_hbm,
                 tab_type, tab_loc, tab_src, tab_tgt,
                 out_hbm, idx_v, rows_v, sem):
    wid = lax.axis_index("s") * _NC + lax.axis_index("c")
    base = wid * _BPW
    idx_refs = (type_hbm, loc_hbm, src_hbm, tgt_hbm)
    tabs = (tab_type, tab_loc, tab_src, tab_tgt)
    for t in range(NUM_TABLES):
        for j in range(_NCHUNK):
            pltpu.sync_copy(idx_refs[t].at[pl.ds(base + j * _CH, _CH)],
                            idx_v.at[j])
        copies = [
            pltpu.async_copy(tabs[t].at[idx_v.at[j]],
                             rows_v.at[pl.ds(j * _CH, _CH)], sem)
            for j in range(_NCHUNK)
        ]
        for c in copies:
            c.wait()
        pltpu.sync_copy(rows_v, out_hbm.at[t, pl.ds(base, _BPW)])


@functools.cache
def _gather():
    return functools.partial(
        pl.kernel,
        mesh=plsc.VectorSubcoreMesh(core_axis_name="c", subcore_axis_name="s"),
        compiler_params=pltpu.CompilerParams(use_tc_tiling_on_sc=False),
        out_type=jax.ShapeDtypeStruct((NUM_TABLES, B, D), jnp.float32),
        scratch_types=[
            pltpu.VMEM((_NCHUNK, _CH), jnp.int32),
            pltpu.VMEM((_BPW, D), jnp.float32),
            pltpu.SemaphoreType.DMA,
        ],
    )(_gather_body)


def _mlp_body(x_ref, ln1g_ref, ln1b_ref, w1_ref, b1_ref, w2_ref, b2_ref,
              ln2g_ref, ln2b_ref, out_ref):
    x = x_ref[...]
    xc = jnp.concatenate([x[0], x[1], x[2], x[3]], axis=-1)
    mu = jnp.mean(xc, axis=-1, keepdims=True)
    xm = xc - mu
    var = jnp.mean(xm * xm, axis=-1, keepdims=True)
    h = xm * lax.rsqrt(var + 1e-5) * ln1g_ref[...] + ln1b_ref[...]
    h = jnp.dot(h, w1_ref[...], preferred_element_type=jnp.float32)
    h = jnp.maximum(h + b1_ref[...], 0.0)
    h = jnp.dot(h, w2_ref[...], preferred_element_type=jnp.float32)
    h = h + b2_ref[...]
    mu2 = jnp.mean(h, axis=-1, keepdims=True)
    hm = h - mu2
    var2 = jnp.mean(hm * hm, axis=-1, keepdims=True)
    out_ref[...] = hm * lax.rsqrt(var2 + 1e-5) * ln2g_ref[...] + ln2b_ref[...]


_R = 2048


def _mlp(x, ln1_g, ln1_b, W1, b1, W2, b2, ln2_g, ln2_b):
    grid = (B // _R,)
    full = lambda shape: pl.BlockSpec(shape, lambda i: (0, 0))
    return pl.pallas_call(
        _mlp_body,
        grid=grid,
        in_specs=[
            pl.BlockSpec((NUM_TABLES, _R, D), lambda i: (0, i, 0)),
            full((1, 4 * D)), full((1, 4 * D)),
            full((4 * D, 2 * D)), full((1, 2 * D)),
            full((2 * D, D)), full((1, D)),
            full((1, D)), full((1, D)),
        ],
        out_specs=pl.BlockSpec((_R, D), lambda i: (i, 0)),
        out_shape=jax.ShapeDtypeStruct((B, D), jnp.float32),
    )(x, ln1_g.reshape(1, -1), ln1_b.reshape(1, -1), W1, b1.reshape(1, -1),
      W2, b2.reshape(1, -1), ln2_g.reshape(1, -1), ln2_b.reshape(1, -1))


def kernel(type_idx, loc_idx, src_idx, tgt_idx, emb_type, emb_loc,
           source_emb, target_emb, ln1_g, ln1_b, W1, b1, W2, b2,
           ln2_g, ln2_b):
    gathered = _gather()(type_idx.astype(jnp.int32), loc_idx.astype(jnp.int32),
                       src_idx.astype(jnp.int32), tgt_idx.astype(jnp.int32),
                       emb_type, emb_loc, source_emb, target_emb)
    return _mlp(gathered, ln1_g, ln1_b, W1, b1, W2, b2, ln2_g, ln2_b)
